# baseline (speedup 1.0000x reference)
import functools

import jax
import jax.numpy as jnp
from jax.experimental import pallas as pl
from jax.experimental.pallas import tpu as pltpu

B, V = 4096, 100
NUM_NAMES, NUM_ROLES, NUM_GROUPS = 100, 8, 8
NAME_D, ROLE_D, GROUP_D = 32, 16, 16
HID, TOK = 128, 64
ROWS = B * V
BLOCK_ROWS = 6400
EPS = 1e-5


def _encoder_kernel(vals_ref, nidx_ref, ridx_ref, gidx_ref,
                    ntab_ref, rtab_ref, gtab_ref,
                    w1v_ref, w1n_ref, w1r_ref, w1g_ref, b1_ref, g1_ref, be1_ref,
                    w2_ref, b2_ref, g2_ref, be2_ref,
                    w3_ref, b3_ref,
                    out_ref, base_ref, mb_ref, cv_ref, vb_ref):
    dot = functools.partial(jax.lax.dot, preferred_element_type=jnp.float32)

    @pl.when(pl.program_id(0) == 0)
    def _prologue():
        ion = jax.lax.broadcasted_iota(jnp.int32, (V, NUM_NAMES), 1)
        ior = jax.lax.broadcasted_iota(jnp.int32, (V, NUM_ROLES), 1)
        iog = jax.lax.broadcasted_iota(jnp.int32, (V, NUM_GROUPS), 1)
        oh_n = (nidx_ref[...] == ion).astype(jnp.float32)
        oh_r = (ridx_ref[...] == ior).astype(jnp.float32)
        oh_g = (gidx_ref[...] == iog).astype(jnp.float32)
        emb_n = dot(oh_n, ntab_ref[...])
        emb_r = dot(oh_r, rtab_ref[...])
        emb_g = dot(oh_g, gtab_ref[...])
        base = (dot(emb_n, w1n_ref[...]) + dot(emb_r, w1r_ref[...])
                + dot(emb_g, w1g_ref[...]) + b1_ref[...])
        w1 = w1v_ref[...]
        mw = jnp.mean(w1, axis=1, keepdims=True)
        mb = jnp.mean(base, axis=1, keepdims=True)
        cv = jnp.mean(base * w1, axis=1, keepdims=True) - mw * mb
        vb = jnp.mean(base * base, axis=1, keepdims=True) - mb * mb
        rr = jax.lax.broadcasted_iota(jnp.int32, (BLOCK_ROWS, V), 0)
        cc = jax.lax.broadcasted_iota(jnp.int32, (BLOCK_ROWS, V), 1)
        sel = (jax.lax.rem(rr, V) == cc).astype(jnp.float32)
        base_ref[...] = dot(sel, base)
        mb_ref[...] = dot(sel, mb)
        cv_ref[...] = dot(sel, cv)
        vb_ref[...] = dot(sel, vb)

    w1 = w1v_ref[...]
    mw = jnp.mean(w1, axis=1, keepdims=True)
    vw = jnp.mean(w1 * w1, axis=1, keepdims=True) - mw * mw

    vcol = vals_ref[...]
    m1 = vcol * mw + mb_ref[...]
    var1 = (vcol * vcol) * vw + 2.0 * (vcol * cv_ref[...]) + vb_ref[...]
    inv1 = jax.lax.rsqrt(var1 + EPS)
    mm = m1 * inv1

    h = vcol * w1 + base_ref[...]
    h = jnp.maximum((h * inv1 - mm) * g1_ref[...] + be1_ref[...], 0.0)
    h = dot(h.astype(jnp.bfloat16), w2_ref[...]) + b2_ref[...]

    m2 = jnp.mean(h, axis=1, keepdims=True)
    v2 = jnp.mean(h * h, axis=1, keepdims=True) - m2 * m2
    inv2 = jax.lax.rsqrt(v2 + EPS)
    h = jnp.maximum((h - m2) * inv2 * g2_ref[...] + be2_ref[...], 0.0)
    out_ref[...] = dot(h.astype(jnp.bfloat16), w3_ref[...]) + b3_ref[...]


def kernel(values, name_idx, role_idx, group_idx, name_table, role_table,
           group_table, W1, b1, g1, be1, W2, b2, g2, be2, W3, b3):
    vals = values.reshape(ROWS, 1)
    grid = ROWS // BLOCK_ROWS

    row_spec = pl.BlockSpec((BLOCK_ROWS, 1), lambda i: (i, 0))
    out_spec = pl.BlockSpec((BLOCK_ROWS, TOK), lambda i: (i, 0))

    def full(shape):
        return pl.BlockSpec(shape, lambda i: (0,) * len(shape))

    out = pl.pallas_call(
        _encoder_kernel,
        grid=(grid,),
        in_specs=[
            row_spec,
            full((V, 1)), full((V, 1)), full((V, 1)),
            full((NUM_NAMES, NAME_D)), full((NUM_ROLES, ROLE_D)),
            full((NUM_GROUPS, GROUP_D)),
            full((1, HID)), full((NAME_D, HID)), full((ROLE_D, HID)),
            full((GROUP_D, HID)), full((1, HID)), full((1, HID)),
            full((1, HID)),
            full((HID, HID)), full((1, HID)), full((1, HID)), full((1, HID)),
            full((HID, TOK)), full((1, TOK)),
        ],
        out_specs=out_spec,
        out_shape=jax.ShapeDtypeStruct((ROWS, TOK), jnp.float32),
        scratch_shapes=[
            pltpu.VMEM((BLOCK_ROWS, HID), jnp.float32),
            pltpu.VMEM((BLOCK_ROWS, 1), jnp.float32),
            pltpu.VMEM((BLOCK_ROWS, 1), jnp.float32),
            pltpu.VMEM((BLOCK_ROWS, 1), jnp.float32),
        ],
        compiler_params=pltpu.CompilerParams(
            dimension_semantics=("arbitrary",),
        ),
    )(
        vals,
        name_idx.reshape(V, 1), role_idx.reshape(V, 1),
        group_idx.reshape(V, 1),
        name_table, role_table, group_table,
        W1[0:1, :], W1[1:1 + NAME_D, :],
        W1[1 + NAME_D:1 + NAME_D + ROLE_D, :],
        W1[1 + NAME_D + ROLE_D:, :],
        b1.reshape(1, HID), g1.reshape(1, HID), be1.reshape(1, HID),
        W2.astype(jnp.bfloat16), b2.reshape(1, HID), g2.reshape(1, HID),
        be2.reshape(1, HID),
        W3.astype(jnp.bfloat16), b3.reshape(1, TOK),
    )
    return out.reshape(B, V, TOK)


# speedup vs baseline: 1.2359x; 1.2359x over previous
import functools

import jax
import jax.numpy as jnp
from jax.experimental import pallas as pl
from jax.experimental.pallas import tpu as pltpu

B, V = 4096, 100
NUM_NAMES, NUM_ROLES, NUM_GROUPS = 100, 8, 8
NAME_D, ROLE_D, GROUP_D = 32, 16, 16
HID, TOK = 128, 64
ROWS = B * V
BLOCK_B = 64
BLOCK_ROWS = BLOCK_B * V
EPS = 1e-5


def _encoder_kernel(vals_ref, nidx_ref, ridx_ref, gidx_ref,
                    ntab_ref, rtab_ref, gtab_ref,
                    w1v_ref, w1n_ref, w1r_ref, w1g_ref, b1_ref, g1_ref, be1_ref,
                    w2_ref, b2_ref, g2_ref, be2_ref,
                    w3_ref, b3_ref,
                    out_ref, base_ref, mb_ref, cv_ref, vb_ref):
    dot = functools.partial(jax.lax.dot, preferred_element_type=jnp.float32)

    @pl.when(pl.program_id(0) == 0)
    def _prologue():
        ion = jax.lax.broadcasted_iota(jnp.int32, (V, NUM_NAMES), 1)
        ior = jax.lax.broadcasted_iota(jnp.int32, (V, NUM_ROLES), 1)
        iog = jax.lax.broadcasted_iota(jnp.int32, (V, NUM_GROUPS), 1)
        oh_n = (nidx_ref[...] == ion).astype(jnp.float32)
        oh_r = (ridx_ref[...] == ior).astype(jnp.float32)
        oh_g = (gidx_ref[...] == iog).astype(jnp.float32)
        emb_n = dot(oh_n, ntab_ref[...])
        emb_r = dot(oh_r, rtab_ref[...])
        emb_g = dot(oh_g, gtab_ref[...])
        base = (dot(emb_n, w1n_ref[...]) + dot(emb_r, w1r_ref[...])
                + dot(emb_g, w1g_ref[...]) + b1_ref[...])
        w1 = w1v_ref[...]
        mw = jnp.mean(w1, axis=1, keepdims=True)
        mb = jnp.mean(base, axis=1, keepdims=True)
        cv = jnp.mean(base * w1, axis=1, keepdims=True) - mw * mb
        vb = jnp.mean(base * base, axis=1, keepdims=True) - mb * mb
        rr = jax.lax.broadcasted_iota(jnp.int32, (BLOCK_ROWS, V), 0)
        cc = jax.lax.broadcasted_iota(jnp.int32, (BLOCK_ROWS, V), 1)
        sel = (jax.lax.rem(rr, V) == cc).astype(jnp.float32)
        base_ref[...] = dot(sel, base)
        mb_ref[...] = dot(sel, mb)
        cv_ref[...] = dot(sel, cv)
        vb_ref[...] = dot(sel, vb)

    w1 = w1v_ref[...]
    mw = jnp.mean(w1, axis=1, keepdims=True)
    vw = jnp.mean(w1 * w1, axis=1, keepdims=True) - mw * mw

    vcol = vals_ref[...]
    m1 = vcol * mw + mb_ref[...]
    var1 = (vcol * vcol) * vw + 2.0 * (vcol * cv_ref[...]) + vb_ref[...]
    inv1 = jax.lax.rsqrt(var1 + EPS)
    mm = m1 * inv1

    h = vcol * w1 + base_ref[...]
    h = jnp.maximum((h * inv1 - mm) * g1_ref[...] + be1_ref[...], 0.0)
    h = dot(h.astype(jnp.bfloat16), w2_ref[...]) + b2_ref[...]

    m2 = jnp.mean(h, axis=1, keepdims=True)
    v2 = jnp.mean(h * h, axis=1, keepdims=True) - m2 * m2
    inv2 = jax.lax.rsqrt(v2 + EPS)
    h = jnp.maximum((h - m2) * inv2 * g2_ref[...] + be2_ref[...], 0.0)
    h = dot(h.astype(jnp.bfloat16), w3_ref[...]) + b3_ref[...]
    out_ref[...] = h.reshape(BLOCK_B, V, TOK)


def kernel(values, name_idx, role_idx, group_idx, name_table, role_table,
           group_table, W1, b1, g1, be1, W2, b2, g2, be2, W3, b3):
    vals = values.reshape(ROWS, 1)
    grid = B // BLOCK_B

    row_spec = pl.BlockSpec((BLOCK_ROWS, 1), lambda i: (i, 0))
    out_spec = pl.BlockSpec((BLOCK_B, V, TOK), lambda i: (i, 0, 0))

    def full(shape):
        return pl.BlockSpec(shape, lambda i: (0,) * len(shape))

    out = pl.pallas_call(
        _encoder_kernel,
        grid=(grid,),
        in_specs=[
            row_spec,
            full((V, 1)), full((V, 1)), full((V, 1)),
            full((NUM_NAMES, NAME_D)), full((NUM_ROLES, ROLE_D)),
            full((NUM_GROUPS, GROUP_D)),
            full((1, HID)), full((NAME_D, HID)), full((ROLE_D, HID)),
            full((GROUP_D, HID)), full((1, HID)), full((1, HID)),
            full((1, HID)),
            full((HID, HID)), full((1, HID)), full((1, HID)), full((1, HID)),
            full((HID, TOK)), full((1, TOK)),
        ],
        out_specs=out_spec,
        out_shape=jax.ShapeDtypeStruct((B, V, TOK), jnp.float32),
        scratch_shapes=[
            pltpu.VMEM((BLOCK_ROWS, HID), jnp.float32),
            pltpu.VMEM((BLOCK_ROWS, 1), jnp.float32),
            pltpu.VMEM((BLOCK_ROWS, 1), jnp.float32),
            pltpu.VMEM((BLOCK_ROWS, 1), jnp.float32),
        ],
        compiler_params=pltpu.CompilerParams(
            dimension_semantics=("arbitrary",),
        ),
    )(
        vals,
        name_idx.reshape(V, 1), role_idx.reshape(V, 1),
        group_idx.reshape(V, 1),
        name_table, role_table, group_table,
        W1[0:1, :], W1[1:1 + NAME_D, :],
        W1[1 + NAME_D:1 + NAME_D + ROLE_D, :],
        W1[1 + NAME_D + ROLE_D:, :],
        b1.reshape(1, HID), g1.reshape(1, HID), be1.reshape(1, HID),
        W2.astype(jnp.bfloat16), b2.reshape(1, HID), g2.reshape(1, HID),
        be2.reshape(1, HID),
        W3.astype(jnp.bfloat16), b3.reshape(1, TOK),
    )
    return out


# speedup vs baseline: 1.9185x; 1.5523x over previous
import functools

import jax
import jax.numpy as jnp
from jax.experimental import pallas as pl
from jax.experimental.pallas import tpu as pltpu

B, V = 4096, 100
NUM_NAMES, NUM_ROLES, NUM_GROUPS = 100, 8, 8
NAME_D, ROLE_D, GROUP_D = 32, 16, 16
HID, TOK = 128, 64
ROWS = B * V
BLOCK_B = 64
BLOCK_ROWS = BLOCK_B * V
EPS = 1e-5


def _encoder_kernel(vals_ref, nidx_ref, ridx_ref, gidx_ref,
                    ntabT_ref, rtabT_ref, gtabT_ref,
                    w1row_ref, w1col_ref, w1nT_ref, w1rT_ref, w1gT_ref,
                    b1c_ref, g1c_ref, be1c_ref,
                    w2T_ref, b2c_ref, g2c_ref, be2c_ref,
                    w3T_ref, b3c_ref,
                    out_ref, baseT_s, mb_s, cv2_s, vbe_s):
    dot = functools.partial(jax.lax.dot, preferred_element_type=jnp.float32)
    ones_row = jnp.full((1, HID), 1.0 / HID, dtype=jnp.float32)

    @pl.when(pl.program_id(0) == 0)
    def _prologue():
        ion = jax.lax.broadcasted_iota(jnp.int32, (NUM_NAMES, V), 0)
        ior = jax.lax.broadcasted_iota(jnp.int32, (NUM_ROLES, V), 0)
        iog = jax.lax.broadcasted_iota(jnp.int32, (NUM_GROUPS, V), 0)
        ohnT = (nidx_ref[...] == ion).astype(jnp.float32)
        ohrT = (ridx_ref[...] == ior).astype(jnp.float32)
        ohgT = (gidx_ref[...] == iog).astype(jnp.float32)
        baseT = (dot(w1nT_ref[...], dot(ntabT_ref[...], ohnT))
                 + dot(w1rT_ref[...], dot(rtabT_ref[...], ohrT))
                 + dot(w1gT_ref[...], dot(gtabT_ref[...], ohgT))
                 + b1c_ref[...])
        w1r = w1row_ref[...]
        mw = jnp.mean(w1r, axis=1, keepdims=True)
        mbv = dot(ones_row, baseT)
        cvv = dot(ones_row * w1r, baseT) - mw * mbv
        vbv = dot(ones_row, baseT * baseT) - mbv * mbv + EPS
        ior2 = jax.lax.broadcasted_iota(jnp.int32, (V, BLOCK_ROWS), 0)
        ioc2 = jax.lax.broadcasted_iota(jnp.int32, (V, BLOCK_ROWS), 1)
        selT = (jax.lax.rem(ioc2, V) == ior2).astype(jnp.float32)
        baseT_s[...] = dot(baseT, selT)
        mb_s[...] = dot(mbv, selT)
        cv2_s[...] = dot(2.0 * cvv, selT)
        vbe_s[...] = dot(vbv, selT)

    w1r = w1row_ref[...]
    mw = jnp.mean(w1r, axis=1, keepdims=True)
    vw = jnp.mean(w1r * w1r, axis=1, keepdims=True) - mw * mw

    vrow = vals_ref[...]
    m1 = vrow * mw + mb_s[...]
    var1 = (vrow * vw + cv2_s[...]) * vrow + vbe_s[...]
    inv1 = jax.lax.rsqrt(var1)
    mm = m1 * inv1

    h = dot(w1col_ref[...], vrow) + baseT_s[...]
    h = jnp.maximum((h * inv1 - mm) * g1c_ref[...] + be1c_ref[...], 0.0)
    h = dot(w2T_ref[...], h.astype(jnp.bfloat16)) + b2c_ref[...]

    m2 = dot(ones_row, h)
    q2 = dot(ones_row, h * h)
    inv2 = jax.lax.rsqrt(q2 - m2 * m2 + EPS)
    mm2 = m2 * inv2
    h = jnp.maximum((h * inv2 - mm2) * g2c_ref[...] + be2c_ref[...], 0.0)
    o = dot(w3T_ref[...], h.astype(jnp.bfloat16)) + b3c_ref[...]
    out_ref[...] = o.T.reshape(BLOCK_B, V, TOK)


def kernel(values, name_idx, role_idx, group_idx, name_table, role_table,
           group_table, W1, b1, g1, be1, W2, b2, g2, be2, W3, b3):
    vals = values.reshape(1, ROWS)
    grid = B // BLOCK_B

    row_spec = pl.BlockSpec((1, BLOCK_ROWS), lambda i: (0, i))
    out_spec = pl.BlockSpec((BLOCK_B, V, TOK), lambda i: (i, 0, 0))

    def full(shape):
        return pl.BlockSpec(shape, lambda i: (0,) * len(shape))

    out = pl.pallas_call(
        _encoder_kernel,
        grid=(grid,),
        in_specs=[
            row_spec,
            full((1, V)), full((1, V)), full((1, V)),
            full((NAME_D, NUM_NAMES)), full((ROLE_D, NUM_ROLES)),
            full((GROUP_D, NUM_GROUPS)),
            full((1, HID)), full((HID, 1)), full((HID, NAME_D)),
            full((HID, ROLE_D)), full((HID, GROUP_D)),
            full((HID, 1)), full((HID, 1)), full((HID, 1)),
            full((HID, HID)), full((HID, 1)), full((HID, 1)), full((HID, 1)),
            full((TOK, HID)), full((TOK, 1)),
        ],
        out_specs=out_spec,
        out_shape=jax.ShapeDtypeStruct((B, V, TOK), jnp.float32),
        scratch_shapes=[
            pltpu.VMEM((HID, BLOCK_ROWS), jnp.float32),
            pltpu.VMEM((1, BLOCK_ROWS), jnp.float32),
            pltpu.VMEM((1, BLOCK_ROWS), jnp.float32),
            pltpu.VMEM((1, BLOCK_ROWS), jnp.float32),
        ],
        compiler_params=pltpu.CompilerParams(
            dimension_semantics=("arbitrary",),
        ),
    )(
        vals,
        name_idx.reshape(1, V), role_idx.reshape(1, V),
        group_idx.reshape(1, V),
        name_table.T, role_table.T, group_table.T,
        W1[0:1, :], W1[0:1, :].T, W1[1:1 + NAME_D, :].T,
        W1[1 + NAME_D:1 + NAME_D + ROLE_D, :].T,
        W1[1 + NAME_D + ROLE_D:, :].T,
        b1.reshape(HID, 1), g1.reshape(HID, 1), be1.reshape(HID, 1),
        W2.T.astype(jnp.bfloat16), b2.reshape(HID, 1), g2.reshape(HID, 1),
        be2.reshape(HID, 1),
        W3.T.astype(jnp.bfloat16), b3.reshape(TOK, 1),
    )
    return out


# speedup vs baseline: 2.0511x; 1.0691x over previous
import functools

import jax
import jax.numpy as jnp
from jax.experimental import pallas as pl
from jax.experimental.pallas import tpu as pltpu

B, V = 4096, 100
NUM_NAMES, NUM_ROLES, NUM_GROUPS = 100, 8, 8
NAME_D, ROLE_D, GROUP_D = 32, 16, 16
HID, TOK = 128, 64
ROWS = B * V
BLOCK_B = 64
BLOCK_ROWS = BLOCK_B * V
EPS = 1e-5


def _encoder_kernel(vals_ref, nidx_ref, ridx_ref, gidx_ref,
                    ntabT_ref, rtabT_ref, gtabT_ref,
                    w1row_ref, w1col_ref, w1nT_ref, w1rT_ref, w1gT_ref,
                    b1c_ref, g1c_ref, be1c_ref,
                    w2T_ref, b2c_ref, g2c_ref, be2c_ref,
                    w3_ref, b3r_ref,
                    out_ref, baseT_s, mb_s, cv2_s, vbe_s):
    dot = functools.partial(jax.lax.dot, preferred_element_type=jnp.float32)
    ones_row = jnp.full((1, HID), 1.0 / HID, dtype=jnp.float32)

    @pl.when(pl.program_id(0) == 0)
    def _prologue():
        ion = jax.lax.broadcasted_iota(jnp.int32, (NUM_NAMES, V), 0)
        ior = jax.lax.broadcasted_iota(jnp.int32, (NUM_ROLES, V), 0)
        iog = jax.lax.broadcasted_iota(jnp.int32, (NUM_GROUPS, V), 0)
        ohnT = (nidx_ref[...] == ion).astype(jnp.float32)
        ohrT = (ridx_ref[...] == ior).astype(jnp.float32)
        ohgT = (gidx_ref[...] == iog).astype(jnp.float32)
        baseT = (dot(w1nT_ref[...], dot(ntabT_ref[...], ohnT))
                 + dot(w1rT_ref[...], dot(rtabT_ref[...], ohrT))
                 + dot(w1gT_ref[...], dot(gtabT_ref[...], ohgT))
                 + b1c_ref[...])
        w1r = w1row_ref[...]
        mw = jnp.mean(w1r, axis=1, keepdims=True)
        mbv = dot(ones_row, baseT)
        cvv = dot(ones_row * w1r, baseT) - mw * mbv
        vbv = dot(ones_row, baseT * baseT) - mbv * mbv + EPS
        ior2 = jax.lax.broadcasted_iota(jnp.int32, (V, BLOCK_ROWS), 0)
        ioc2 = jax.lax.broadcasted_iota(jnp.int32, (V, BLOCK_ROWS), 1)
        selT = (jax.lax.rem(ioc2, V) == ior2).astype(jnp.float32)
        baseT_s[...] = dot(baseT, selT)
        mb_s[...] = dot(mbv, selT)
        cv2_s[...] = dot(2.0 * cvv, selT)
        vbe_s[...] = dot(vbv, selT)

    w1r = w1row_ref[...]
    mw = jnp.mean(w1r, axis=1, keepdims=True)
    vw = jnp.mean(w1r * w1r, axis=1, keepdims=True) - mw * mw

    vrow = vals_ref[...]
    m1 = vrow * mw + mb_s[...]
    var1 = (vrow * vw + cv2_s[...]) * vrow + vbe_s[...]
    inv1 = jax.lax.rsqrt(var1)
    mm = m1 * inv1

    h = dot(w1col_ref[...], vrow) + baseT_s[...]
    h = jnp.maximum((h * inv1 - mm) * g1c_ref[...] + be1c_ref[...], 0.0)
    h = dot(w2T_ref[...], h.astype(jnp.bfloat16)) + b2c_ref[...]

    m2 = dot(ones_row, h)
    q2 = dot(ones_row, h * h)
    inv2 = jax.lax.rsqrt(q2 - m2 * m2 + EPS)
    mm2 = m2 * inv2
    h = jnp.maximum((h * inv2 - mm2) * g2c_ref[...] + be2c_ref[...], 0.0)
    o = jax.lax.dot_general(h.astype(jnp.bfloat16), w3_ref[...],
                            (((0,), (0,)), ((), ())),
                            preferred_element_type=jnp.float32)
    out_ref[...] = (o + b3r_ref[...]).reshape(BLOCK_B, V, TOK)


def kernel(values, name_idx, role_idx, group_idx, name_table, role_table,
           group_table, W1, b1, g1, be1, W2, b2, g2, be2, W3, b3):
    vals = values.reshape(1, ROWS)
    grid = B // BLOCK_B

    row_spec = pl.BlockSpec((1, BLOCK_ROWS), lambda i: (0, i))
    out_spec = pl.BlockSpec((BLOCK_B, V, TOK), lambda i: (i, 0, 0))

    def full(shape):
        return pl.BlockSpec(shape, lambda i: (0,) * len(shape))

    out = pl.pallas_call(
        _encoder_kernel,
        grid=(grid,),
        in_specs=[
            row_spec,
            full((1, V)), full((1, V)), full((1, V)),
            full((NAME_D, NUM_NAMES)), full((ROLE_D, NUM_ROLES)),
            full((GROUP_D, NUM_GROUPS)),
            full((1, HID)), full((HID, 1)), full((HID, NAME_D)),
            full((HID, ROLE_D)), full((HID, GROUP_D)),
            full((HID, 1)), full((HID, 1)), full((HID, 1)),
            full((HID, HID)), full((HID, 1)), full((HID, 1)), full((HID, 1)),
            full((HID, TOK)), full((1, TOK)),
        ],
        out_specs=out_spec,
        out_shape=jax.ShapeDtypeStruct((B, V, TOK), jnp.float32),
        scratch_shapes=[
            pltpu.VMEM((HID, BLOCK_ROWS), jnp.float32),
            pltpu.VMEM((1, BLOCK_ROWS), jnp.float32),
            pltpu.VMEM((1, BLOCK_ROWS), jnp.float32),
            pltpu.VMEM((1, BLOCK_ROWS), jnp.float32),
        ],
        compiler_params=pltpu.CompilerParams(
            dimension_semantics=("arbitrary",),
        ),
    )(
        vals,
        name_idx.reshape(1, V), role_idx.reshape(1, V),
        group_idx.reshape(1, V),
        name_table.T, role_table.T, group_table.T,
        W1[0:1, :], W1[0:1, :].T, W1[1:1 + NAME_D, :].T,
        W1[1 + NAME_D:1 + NAME_D + ROLE_D, :].T,
        W1[1 + NAME_D + ROLE_D:, :].T,
        b1.reshape(HID, 1), g1.reshape(HID, 1), be1.reshape(HID, 1),
        W2.T.astype(jnp.bfloat16), b2.reshape(HID, 1), g2.reshape(HID, 1),
        be2.reshape(HID, 1),
        W3.astype(jnp.bfloat16), b3.reshape(1, TOK),
    )
    return out
